# Initial kernel scaffold; baseline (speedup 1.0000x reference)
import jax
import jax.numpy as jnp
from jax.experimental import pallas as pl


def kernel(x, soft_labels, edge_index, edge_attr, W0, b0, W1, b1):
    raise NotImplementedError("write your pallas kernel here")


# speedup vs baseline: 5.6309x; 5.6309x over previous
import functools

import jax
import jax.numpy as jnp
from jax import lax
from jax.experimental import pallas as pl
from jax.experimental.pallas import tpu as pltpu
from jax.experimental.pallas import tpu_sc as plsc

N = 10000
E = 320000
NC = 2
NS = 16
NW = NC * NS
EPW = E // NW
C = 80
NCH = EPW // C
RPT = N // NS
ND = 10240
RPTD = ND // NS
L = 16


def _make_scatter(D, with_deg):
    out_type = [jax.ShapeDtypeStruct((NC, ND, D), jnp.float32)]
    if with_deg:
        out_type.append(jax.ShapeDtypeStruct((NC, ND), jnp.float32))
    scratch = [
        pltpu.VMEM((NCH, C), jnp.int32),
        pltpu.VMEM((NCH, C), jnp.int32),
        pltpu.VMEM((NCH, C), jnp.float32),
        pltpu.VMEM((C, D), jnp.float32),
        pltpu.VMEM_SHARED((ND, D), jnp.float32),
    ]
    if with_deg:
        scratch.append(pltpu.VMEM_SHARED((ND,), jnp.float32))
    scratch.append(pltpu.SemaphoreType.DMA)
    mesh = plsc.VectorSubcoreMesh(core_axis_name="c", subcore_axis_name="s")

    def body(row_hbm, col_hbm, ea_hbm, src_hbm, z2_hbm, *rest):
        if with_deg:
            (z1_hbm, out_hbm, deg_hbm,
             row_v, col_v, ea_v, rows_v, acc, acc_deg, sem) = rest
        else:
            out_hbm, row_v, col_v, ea_v, rows_v, acc, sem = rest
        cid = lax.axis_index("c")
        sid = lax.axis_index("s")
        wid = sid * NC + cid
        pltpu.sync_copy(z2_hbm, acc.at[pl.ds(sid * RPTD, RPTD)])
        if with_deg:
            pltpu.sync_copy(z1_hbm, acc_deg.at[pl.ds(sid * RPTD, RPTD)])
        pltpu.sync_copy(row_hbm.at[wid], row_v)
        pltpu.sync_copy(col_hbm.at[wid], col_v)
        pltpu.sync_copy(ea_hbm.at[wid], ea_v)
        plsc.subcore_barrier()

        def chunk(j, carry):
            pltpu.async_copy(src_hbm.at[col_v.at[j]], rows_v, sem).wait()
            if with_deg:
                pltpu.sync_copy(ea_v.at[j], acc_deg.at[row_v.at[j]], add=True)

            def group(g, carry2):
                eav = ea_v[j, pl.ds(g * L, L)]
                for el in range(L):
                    sv = jnp.full((L,), eav[el], jnp.float32)
                    e = g * L + el
                    for d in range(D // L):
                        sl = pl.ds(d * L, L)
                        rows_v[e, sl] = rows_v[e, sl] * sv
                return carry2

            lax.fori_loop(0, C // L, group, 0)
            pltpu.sync_copy(rows_v, acc.at[row_v.at[j]], add=True)
            return carry

        lax.fori_loop(0, NCH, chunk, 0)
        plsc.subcore_barrier()

        sl = pl.ds(sid * RPTD, RPTD)
        pltpu.sync_copy(acc.at[sl], out_hbm.at[cid, sl])
        if with_deg:
            sld = pl.ds(sid * RPTD, RPTD)
            pltpu.sync_copy(acc_deg.at[sld], deg_hbm.at[cid, sld])

    return pl.kernel(body, out_type=tuple(out_type) if with_deg else out_type[0],
                     mesh=mesh, scratch_types=scratch,
                     compiler_params=pltpu.CompilerParams(use_tc_tiling_on_sc=False))


_scatter64d = _make_scatter(64, True)
_scatter16 = _make_scatter(16, False)
_scatter48 = _make_scatter(48, False)


_BM = 1000


def _mm0_body(x_ref, w_ref, o_ref):
    o_ref[...] = jnp.dot(x_ref[...], w_ref[...],
                         preferred_element_type=jnp.float32)


def _matmul0(x, w):
    m, k = x.shape
    n = w.shape[1]
    return pl.pallas_call(
        _mm0_body,
        grid=(m // _BM,),
        in_specs=[pl.BlockSpec((_BM, k), lambda i: (i, 0)),
                  pl.BlockSpec((k, n), lambda i: (0, 0))],
        out_specs=pl.BlockSpec((_BM, n), lambda i: (i, 0)),
        out_shape=jax.ShapeDtypeStruct((m, n), jnp.float32),
    )(x, w)


def _dinv(dp0_ref, dp1_ref):
    deg = dp0_ref[...] + dp1_ref[...]
    return jnp.where(deg == 0.0, 0.0, 1.0 / deg)


def _hidden_mm_body(dp0_ref, dp1_ref, a_ref, b_ref, b0_ref, w_ref, o_ref):
    h = _dinv(dp0_ref, dp1_ref) * (a_ref[...] + b_ref[...]) + b0_ref[...]
    h = jnp.maximum(h, 0.0)
    o_ref[...] = jnp.dot(h, w_ref[...], preferred_element_type=jnp.float32)


def _hidden_mm(dp0, dp1, a, b, b0, w):
    n_out = w.shape[1]
    return pl.pallas_call(
        _hidden_mm_body,
        grid=(N // _BM,),
        in_specs=[pl.BlockSpec((_BM, 1), lambda i: (i, 0)),
                  pl.BlockSpec((_BM, 1), lambda i: (i, 0)),
                  pl.BlockSpec((_BM, 128), lambda i: (i, 0)),
                  pl.BlockSpec((_BM, 128), lambda i: (i, 0)),
                  pl.BlockSpec((1, 128), lambda i: (0, 0)),
                  pl.BlockSpec((128, n_out), lambda i: (0, 0))],
        out_specs=pl.BlockSpec((_BM, n_out), lambda i: (i, 0)),
        out_shape=jax.ShapeDtypeStruct((N, n_out), jnp.float32),
    )(dp0, dp1, a, b, b0, w)


def _out_l1_body(dp0_ref, dp1_ref, a_ref, b_ref, bias_ref, o_ref, l_ref):
    t = _dinv(dp0_ref, dp1_ref) * (a_ref[...] + b_ref[...]) + bias_ref[...]
    o_ref[...] = t[:, :40]
    l_ref[...] = jnp.concatenate(
        [t[:, 40:], jnp.zeros((t.shape[0], 8), jnp.float32)], axis=1)


def _out_l1(dp0, dp1, a, b, bias):
    return pl.pallas_call(
        _out_l1_body,
        grid=(N // _BM,),
        in_specs=[pl.BlockSpec((_BM, 1), lambda i: (i, 0)),
                  pl.BlockSpec((_BM, 1), lambda i: (i, 0)),
                  pl.BlockSpec((_BM, 80), lambda i: (i, 0)),
                  pl.BlockSpec((_BM, 80), lambda i: (i, 0)),
                  pl.BlockSpec((1, 80), lambda i: (0, 0))],
        out_specs=[pl.BlockSpec((_BM, 40), lambda i: (i, 0)),
                   pl.BlockSpec((_BM, 48), lambda i: (i, 0))],
        out_shape=[jax.ShapeDtypeStruct((N, 40), jnp.float32),
                   jax.ShapeDtypeStruct((N, 48), jnp.float32)],
    )(dp0, dp1, a, b, bias)


def _combine_body(dp0_ref, dp1_ref, a_ref, b_ref, o_ref):
    o_ref[...] = _dinv(dp0_ref, dp1_ref) * (a_ref[...] + b_ref[...])


def _combine48(dp0, dp1, a, b):
    return pl.pallas_call(
        _combine_body,
        grid=(N // _BM,),
        in_specs=[pl.BlockSpec((_BM, 1), lambda i: (i, 0)),
                  pl.BlockSpec((_BM, 1), lambda i: (i, 0)),
                  pl.BlockSpec((_BM, 48), lambda i: (i, 0)),
                  pl.BlockSpec((_BM, 48), lambda i: (i, 0))],
        out_specs=pl.BlockSpec((_BM, 48), lambda i: (i, 0)),
        out_shape=jax.ShapeDtypeStruct((N, 48), jnp.float32),
    )(dp0, dp1, a, b)


def kernel(x, soft_labels, edge_index, edge_attr, W0, b0, W1, b1):
    row = edge_index[0].astype(jnp.int32).reshape(NW, NCH, C)
    col = edge_index[1].astype(jnp.int32).reshape(NW, NCH, C)
    ea = edge_attr.reshape(NW, NCH, C)
    z64 = jnp.zeros((RPTD, 64), jnp.float32)
    z16 = jnp.zeros((RPTD, 16), jnp.float32)
    z48 = jnp.zeros((RPTD, 48), jnp.float32)
    z1 = jnp.zeros((RPTD,), jnp.float32)
    b1pad = jnp.concatenate([b1, jnp.zeros((40,), jnp.float32)]).reshape(1, 80)

    xw = _matmul0(x, W0)
    s1a, degp = _scatter64d(row, col, ea, xw[:, :64], z64, z1)
    s1b, _ = _scatter64d(row, col, ea, xw[:, 64:], z64, z1)
    dp0 = degp[0, :N].reshape(N, 1)
    dp1 = degp[1, :N].reshape(N, 1)
    s1_0 = jnp.concatenate([s1a[0, :N], s1b[0, :N]], axis=1)
    s1_1 = jnp.concatenate([s1a[1, :N], s1b[1, :N]], axis=1)
    hw1 = _hidden_mm(dp0, dp1, s1_0, s1_1, b0.reshape(1, 128), W1)
    src80 = jnp.concatenate([hw1, soft_labels], axis=1)
    s2a, _ = _scatter64d(row, col, ea, src80[:, :64], z64, z1)
    s2b = _scatter16(row, col, ea, src80[:, 64:], z16)
    s2_0 = jnp.concatenate([s2a[0, :N], s2b[0, :N]], axis=1)
    s2_1 = jnp.concatenate([s2a[1, :N], s2b[1, :N]], axis=1)
    out, l1 = _out_l1(dp0, dp1, s2_0, s2_1, b1pad)
    s3 = _scatter48(row, col, ea, l1, z48)
    l2 = _combine48(dp0, dp1, s3[0, :N], s3[1, :N])
    s4 = _scatter48(row, col, ea, l2, z48)
    l3 = _combine48(dp0, dp1, s4[0, :N], s4[1, :N])
    return out, l3[:, :40]
